# Initial kernel scaffold; baseline (speedup 1.0000x reference)
import jax
import jax.numpy as jnp
from jax.experimental import pallas as pl


def kernel(x, ve, cos, sin, window_size, Wq, Wk, Wv, Wo, Wg, Wr, Wfc_s, Wproj_s, W1, W2):
    raise NotImplementedError("write your pallas kernel here")


# speedup vs baseline: 1.2565x; 1.2565x over previous
import numpy as np
import jax
import jax.numpy as jnp
from jax.experimental import pallas as pl

_EPS = 1.1920929e-07
_T, _C, _H, _KVH, _HD, _E = 2048, 768, 12, 4, 64, 8
_HALF = _HD // 2
_BT = 256
_TQ = 256


def _rms(x):
    return x * jax.lax.rsqrt(jnp.mean(jnp.square(x), axis=-1, keepdims=True) + _EPS)


def _prep_body(x_ref, ve_ref, c12_ref, s12_ref, wq_ref, wk_ref, wv_ref, wg_ref,
               q_ref, k_ref, v_ref):
    x = x_ref[...]
    xn = _rms(x)
    q = jnp.dot(xn, wq_ref[...], preferred_element_type=jnp.float32)
    k = jnp.dot(xn, wk_ref[...], preferred_element_type=jnp.float32)
    v = jnp.dot(xn, wv_ref[...], preferred_element_type=jnp.float32)
    gate = 2.0 * jax.nn.sigmoid(
        jnp.dot(xn[:, :32], wg_ref[...], preferred_element_type=jnp.float32))
    rows = jax.lax.broadcasted_iota(jnp.int32, (_KVH, _KVH * _HD), 0)
    cols = jax.lax.broadcasted_iota(jnp.int32, (_KVH, _KVH * _HD), 1)
    expand = (cols // _HD == rows).astype(jnp.float32)
    g64 = jnp.dot(gate, expand, preferred_element_type=jnp.float32)
    v_ref[...] = v + g64 * ve_ref[...]
    c12 = c12_ref[...]
    s12 = s12_ref[...]
    nq1 = _H * _HALF
    q1 = q[:, :nq1]
    q2 = q[:, nq1:]
    q_ref[...] = jnp.concatenate([q1 * c12 + q2 * s12, q2 * c12 - q1 * s12], axis=1)
    nk1 = _KVH * _HALF
    c4 = c12[:, :nk1]
    s4 = s12[:, :nk1]
    k1 = k[:, :nk1]
    k2 = k[:, nk1:]
    k_ref[...] = jnp.concatenate([k1 * c4 + k2 * s4, k2 * c4 - k1 * s4], axis=1)


def _attn_body(q_ref, k_ref, v_ref, o_ref):
    iq = pl.program_id(1)
    q = _rms(q_ref[0]) * (1.0 / np.sqrt(_HD))
    k = _rms(k_ref[0])
    s = jax.lax.dot_general(q, k, (((1,), (1,)), ((), ())),
                            preferred_element_type=jnp.float32)
    row = jax.lax.broadcasted_iota(jnp.int32, s.shape, 0) + iq * _TQ
    col = jax.lax.broadcasted_iota(jnp.int32, s.shape, 1)
    s = jnp.where(col <= row, s, -1e30)
    m = jnp.max(s, axis=-1, keepdims=True)
    p = jnp.exp(s - m)
    l = jnp.sum(p, axis=-1, keepdims=True)
    o = jnp.dot(p, v_ref[0], preferred_element_type=jnp.float32)
    o_ref[0] = o / l


def _post_body(x_ref, y_ref, wo_ref, wfc_ref, wproj_ref, wrt_ref,
               base_ref, xn2_ref, g_ref):
    attn = jnp.dot(y_ref[...], wo_ref[...], preferred_element_type=jnp.float32)
    xnew = x_ref[...] + attn
    xn2 = _rms(xnew)
    xn2_ref[...] = xn2
    hs = jnp.maximum(jnp.dot(xn2, wfc_ref[...], preferred_element_type=jnp.float32), 0.0)
    shared = jnp.dot(hs * hs, wproj_ref[...], preferred_element_type=jnp.float32)
    base_ref[...] = xnew + shared
    r = jax.nn.sigmoid(jnp.dot(xn2, wrt_ref[...], preferred_element_type=jnp.float32))
    lane = jax.lax.broadcasted_iota(jnp.int32, r.shape, 1)
    m1 = jnp.max(r, axis=-1, keepdims=True)
    i1 = jnp.min(jnp.where(r == m1, lane, _E), axis=-1, keepdims=True)
    mask1 = lane == i1
    r2 = jnp.where(mask1, -1.0, r)
    m2 = jnp.max(r2, axis=-1, keepdims=True)
    i2 = jnp.min(jnp.where(r2 == m2, lane, _E), axis=-1, keepdims=True)
    mask2 = lane == i2
    g_ref[...] = (jnp.where(mask1, m1, 0.0) + jnp.where(mask2, m2, 0.0)) / (
        m1 + m2 + 1e-20)


def _moe_body(xn2_ref, g_ref, base_ref, w1_ref, w2_ref, out_ref):
    e = pl.program_id(0)
    h = jnp.maximum(jnp.dot(xn2_ref[...], w1_ref[0], preferred_element_type=jnp.float32), 0.0)
    o = jnp.dot(h * h, w2_ref[0], preferred_element_type=jnp.float32)
    g = g_ref[...]
    lane = jax.lax.broadcasted_iota(jnp.int32, g.shape, 1)
    ge = jnp.sum(jnp.where(lane == e, g, 0.0), axis=-1, keepdims=True)
    contrib = ge * o

    @pl.when(e == 0)
    def _():
        out_ref[...] = base_ref[...] + contrib

    @pl.when(e != 0)
    def _():
        out_ref[...] += contrib


def kernel(x, ve, cos, sin, window_size, Wq, Wk, Wv, Wo, Wg, Wr, Wfc_s, Wproj_s, W1, W2):
    B, T, C = x.shape
    assert (B, T, C) == (1, _T, _C)
    xf = x.reshape(_T, _C)
    vef = ve.reshape(_T, _KVH * _HD)
    cosf = cos.reshape(_T, _HALF)
    sinf = sin.reshape(_T, _HALF)
    c12 = jnp.tile(cosf, (1, _H))
    s12 = jnp.tile(sinf, (1, _H))
    permq = np.concatenate(
        [np.arange(_HALF) + h * _HD for h in range(_H)]
        + [np.arange(_HALF) + h * _HD + _HALF for h in range(_H)])
    permk = np.concatenate(
        [np.arange(_HALF) + h * _HD for h in range(_KVH)]
        + [np.arange(_HALF) + h * _HD + _HALF for h in range(_KVH)])
    Wqp = Wq[:, permq]
    Wkp = Wk[:, permk]

    nt = _T // _BT
    full = lambda shape: pl.BlockSpec(shape, lambda i: (0,) * len(shape))
    qp, kp, vg = pl.pallas_call(
        _prep_body,
        grid=(nt,),
        in_specs=[
            pl.BlockSpec((_BT, _C), lambda i: (i, 0)),
            pl.BlockSpec((_BT, _KVH * _HD), lambda i: (i, 0)),
            pl.BlockSpec((_BT, _H * _HALF), lambda i: (i, 0)),
            pl.BlockSpec((_BT, _H * _HALF), lambda i: (i, 0)),
            full((_C, _H * _HD)),
            full((_C, _KVH * _HD)),
            full((_C, _KVH * _HD)),
            full((32, _KVH)),
        ],
        out_specs=[
            pl.BlockSpec((_BT, _C), lambda i: (i, 0)),
            pl.BlockSpec((_BT, _KVH * _HD), lambda i: (i, 0)),
            pl.BlockSpec((_BT, _KVH * _HD), lambda i: (i, 0)),
        ],
        out_shape=[
            jax.ShapeDtypeStruct((_T, _C), jnp.float32),
            jax.ShapeDtypeStruct((_T, _KVH * _HD), jnp.float32),
            jax.ShapeDtypeStruct((_T, _KVH * _HD), jnp.float32),
        ],
    )(xf, vef, c12, s12, Wqp, Wkp, Wv, Wg)

    qh = qp.reshape(_T, 2, _H, _HALF).transpose(2, 0, 1, 3).reshape(_H, _T, _HD)
    kh = kp.reshape(_T, 2, _KVH, _HALF).transpose(2, 0, 1, 3).reshape(_KVH, _T, _HD)
    vh = vg.reshape(_T, _KVH, _HD).transpose(1, 0, 2)

    rep = _H // _KVH
    oh = pl.pallas_call(
        _attn_body,
        grid=(_H, _T // _TQ),
        in_specs=[
            pl.BlockSpec((1, _TQ, _HD), lambda h, i: (h, i, 0)),
            pl.BlockSpec((1, _T, _HD), lambda h, i: (h // rep, 0, 0)),
            pl.BlockSpec((1, _T, _HD), lambda h, i: (h // rep, 0, 0)),
        ],
        out_specs=pl.BlockSpec((1, _TQ, _HD), lambda h, i: (h, i, 0)),
        out_shape=jax.ShapeDtypeStruct((_H, _T, _HD), jnp.float32),
    )(qh, kh, vh)

    y = oh.transpose(1, 0, 2).reshape(_T, _C)

    base, xn2, g = pl.pallas_call(
        _post_body,
        grid=(nt,),
        in_specs=[
            pl.BlockSpec((_BT, _C), lambda i: (i, 0)),
            pl.BlockSpec((_BT, _C), lambda i: (i, 0)),
            full((_C, _C)),
            full((_C, _C)),
            full((_C, _C)),
            full((_C, _E)),
        ],
        out_specs=[
            pl.BlockSpec((_BT, _C), lambda i: (i, 0)),
            pl.BlockSpec((_BT, _C), lambda i: (i, 0)),
            pl.BlockSpec((_BT, _E), lambda i: (i, 0)),
        ],
        out_shape=[
            jax.ShapeDtypeStruct((_T, _C), jnp.float32),
            jax.ShapeDtypeStruct((_T, _C), jnp.float32),
            jax.ShapeDtypeStruct((_T, _E), jnp.float32),
        ],
    )(xf, y, Wo, Wfc_s, Wproj_s, Wr.T)

    out = pl.pallas_call(
        _moe_body,
        grid=(_E,),
        in_specs=[
            pl.BlockSpec((_T, _C), lambda e: (0, 0)),
            pl.BlockSpec((_T, _E), lambda e: (0, 0)),
            pl.BlockSpec((_T, _C), lambda e: (0, 0)),
            pl.BlockSpec((1, _C, _C), lambda e: (e, 0, 0)),
            pl.BlockSpec((1, _C, _C), lambda e: (e, 0, 0)),
        ],
        out_specs=pl.BlockSpec((_T, _C), lambda e: (0, 0)),
        out_shape=jax.ShapeDtypeStruct((_T, _C), jnp.float32),
    )(xn2, g, base, W1, W2)

    return out.reshape(1, _T, _C)
